# baseline (speedup 1.0000x reference)
import functools

import jax
import jax.numpy as jnp
from jax import lax
from jax.experimental import pallas as pl


def _softmax_body(x_ref, o_ref):
    x = x_ref[...]
    m = jnp.max(x, axis=-1, keepdims=True)
    e = jnp.exp(x - m)
    s = jnp.sum(e, axis=-1, keepdims=True)
    o_ref[...] = e / s


def _route_body(idx_ref, table_ref, o_ref, *, n):
    idx = idx_ref[0, 0, :]
    blk = idx.shape[0]
    onehot = (lax.broadcasted_iota(jnp.int32, (n, blk), 0)
              == idx[None, :]).astype(jnp.float32)
    o_ref[...] = lax.dot_general(
        table_ref[...], onehot, (((0,), (0,)), ((), ())),
        preferred_element_type=jnp.float32)


def kernel(inputs_idx, params):
    B = inputs_idx.shape[0]
    N, D = params.shape

    sm_table = pl.pallas_call(
        _softmax_body,
        out_shape=jax.ShapeDtypeStruct((N, D), jnp.float32),
    )(params)

    idx32 = inputs_idx.astype(jnp.int32)
    BLK = 4096
    idx3 = idx32.reshape(B // BLK, 1, BLK)

    out_t = pl.pallas_call(
        functools.partial(_route_body, n=N),
        grid=(B // BLK,),
        in_specs=[
            pl.BlockSpec((1, 1, BLK), lambda i: (i, 0, 0)),
            pl.BlockSpec((N, D), lambda i: (0, 0)),
        ],
        out_specs=pl.BlockSpec((D, BLK), lambda i: (0, i)),
        out_shape=jax.ShapeDtypeStruct((D, B), jnp.float32),
    )(idx3, sm_table)

    return out_t.T


# speedup vs baseline: 1.1592x; 1.1592x over previous
import functools

import jax
import jax.numpy as jnp
from jax import lax
from jax.experimental import pallas as pl
from jax.experimental.pallas import tpu as pltpu


def _route_body(idx_ref, x_ref, o_ref, sm_ref, *, n):
    @pl.when(pl.program_id(0) == 0)
    def _():
        x = x_ref[...]
        m = jnp.max(x, axis=-1, keepdims=True)
        e = jnp.exp(x - m)
        s = jnp.sum(e, axis=-1, keepdims=True)
        sm_ref[...] = e / s

    idx = idx_ref[0, 0, :]
    blk = idx.shape[0]
    onehot = (lax.broadcasted_iota(jnp.int32, (n, blk), 0)
              == idx[None, :]).astype(jnp.float32)
    o_ref[...] = lax.dot_general(
        sm_ref[...], onehot, (((0,), (0,)), ((), ())),
        preferred_element_type=jnp.float32)


def kernel(inputs_idx, params):
    B = inputs_idx.shape[0]
    N, D = params.shape

    idx32 = inputs_idx.astype(jnp.int32)
    BLK = 2048
    idx3 = idx32.reshape(B // BLK, 1, BLK)

    out_t = pl.pallas_call(
        functools.partial(_route_body, n=N),
        grid=(B // BLK,),
        in_specs=[
            pl.BlockSpec((1, 1, BLK), lambda i: (i, 0, 0)),
            pl.BlockSpec((N, D), lambda i: (0, 0)),
        ],
        out_specs=pl.BlockSpec((D, BLK), lambda i: (0, i)),
        out_shape=jax.ShapeDtypeStruct((D, B), jnp.float32),
        scratch_shapes=[pltpu.VMEM((N, D), jnp.float32)],
    )(idx3, params)

    return out_t.T
